# baseline (speedup 1.0000x reference)
import jax
import jax.numpy as jnp
import numpy as np
from jax.experimental import pallas as pl
from jax.experimental.pallas import tpu as pltpu

NUM_SENSORS = 8192
TOP_K = 128
FEAT = 2048
EPS = 1e-6
BF = 256


_ROTATIONS = ((13, 15, 26, 6), (17, 29, 16, 24))


def _rotl(x, r):
    return (x << np.uint32(r)) | (x >> np.uint32(32 - r))


def _threefry2x32(k0, k1, x0, x1):
    ks = (np.uint32(k0), np.uint32(k1),
          np.uint32(k0) ^ np.uint32(k1) ^ np.uint32(0x1BD11BDA))
    x0 = x0 + ks[0]
    x1 = x1 + ks[1]
    for i in range(5):
        for r in _ROTATIONS[i % 2]:
            x0 = x0 + x1
            x1 = _rotl(x1, r)
            x1 = x0 ^ x1
        x0 = x0 + ks[(i + 1) % 3]
        x1 = x1 + ks[(i + 2) % 3] + np.uint32(i + 1)
    return x0, x1


def _np_fold_in(k0, k1, data):
    a, b = _threefry2x32(k0, k1,
                         np.uint32(data >> 32), np.uint32(data & 0xFFFFFFFF))
    return int(a), int(b)


def _np_uniform(k0, k1, shape, minval=0.0, maxval=1.0):
    i = np.arange(int(np.prod(shape)), dtype=np.uint64)
    hi = (i >> np.uint64(32)).astype(np.uint32)
    lo = (i & np.uint64(0xFFFFFFFF)).astype(np.uint32)
    x0, x1 = _threefry2x32(k0, k1, hi, lo)
    bits = x0 ^ x1
    floats = ((bits >> np.uint32(9)) | np.uint32(0x3F800000)).view(np.float32) \
        - np.float32(1.0)
    u = floats * np.float32(maxval - minval) + np.float32(minval)
    return np.maximum(np.float32(minval), u).reshape(shape)


def _rng_consts():
    old = np.seterr(over="ignore")
    try:
        gk = _np_fold_in(0, 0, 1)
        U = _np_uniform(gk[0], gk[1], (NUM_SENSORS, TOP_K))
        e1 = 1.0 / (np.float32(EPS) - np.log(U + np.float32(EPS)))
        ik = _np_fold_in(0, 0, 2)
        tiny = float(np.finfo(np.float32).tiny)
        Ug = _np_uniform(ik[0], ik[1], (TOP_K, NUM_SENSORS), minval=tiny)
        e2t = np.ascontiguousarray((1.0 / (-np.log(Ug))).T)
        return e1.astype(np.float32), e2t.astype(np.float32)
    finally:
        np.seterr(**old)


_E1, _E2T = _rng_consts()


def _body(beta_ref, alpha_ref, e1_ref, e2t_ref, x_ref,
          z_ref, p_ref, idx_ref, w_ref, s_ref):
    j = pl.program_id(0)

    @pl.when(j == 0)
    def _softmax_weights():
        alpha = alpha_ref[...]
        y = 50.0 * alpha
        sp = (y + jnp.log1p(jnp.exp(-y))) * (1.0 / 50.0) + EPS
        base = sp * e1_ref[...]
        beta = beta_ref[0, 0]
        unnorm = jax.lax.cond(
            beta == 1.0,
            lambda b: b,
            lambda b: jnp.exp2(jnp.log2(b) * (1.0 / beta)),
            base)
        w = unnorm / jnp.sum(unnorm, axis=0, keepdims=True)
        w_ref[...] = w.astype(jnp.bfloat16)
        csum = jnp.sum(alpha, axis=0, keepdims=True)
        inv0 = 1.0 / (csum + EPS)
        rs = csum * inv0
        s_ref[0:1, :] = inv0
        s_ref[1:2, :] = inv0 / (rs + EPS)

    @pl.when(j == 1)
    def _p_out():
        p_ref[...] = alpha_ref[...] * s_ref[0:1, :]

    @pl.when(j == 2)
    def _indices():
        val = (alpha_ref[...] * s_ref[1:2, :] + EPS) * e2t_ref[...]
        mx = jnp.max(val, axis=0, keepdims=True)
        iota = jax.lax.broadcasted_iota(jnp.int32, val.shape, 0)
        idx_ref[...] = jnp.min(
            jnp.where(val == mx, iota, NUM_SENSORS), axis=0, keepdims=True)

    z_ref[...] = jax.lax.dot_general(
        w_ref[...], x_ref[...].astype(jnp.bfloat16),
        dimension_numbers=(((0,), (0,)), ((), ())),
        preferred_element_type=jnp.float32)


def kernel(X, beta, alpha):
    e1, e2t = _E1, _E2T
    beta_arr = jnp.asarray(beta, jnp.float32).reshape(1, 1)
    grid = (FEAT // BF,)
    Z, p, idx = pl.pallas_call(
        _body,
        grid=grid,
        in_specs=[
            pl.BlockSpec(memory_space=pltpu.SMEM),
            pl.BlockSpec((NUM_SENSORS, TOP_K), lambda j: (0, 0)),
            pl.BlockSpec((NUM_SENSORS, TOP_K), lambda j: (0, 0)),
            pl.BlockSpec((NUM_SENSORS, TOP_K), lambda j: (0, 0)),
            pl.BlockSpec((NUM_SENSORS, BF), lambda j: (0, j)),
        ],
        out_specs=[
            pl.BlockSpec((TOP_K, BF), lambda j: (0, j)),
            pl.BlockSpec((NUM_SENSORS, TOP_K), lambda j: (0, 0)),
            pl.BlockSpec((1, TOP_K), lambda j: (0, 0)),
        ],
        out_shape=[
            jax.ShapeDtypeStruct((TOP_K, FEAT), jnp.float32),
            jax.ShapeDtypeStruct((NUM_SENSORS, TOP_K), jnp.float32),
            jax.ShapeDtypeStruct((1, TOP_K), jnp.int32),
        ],
        scratch_shapes=[pltpu.VMEM((NUM_SENSORS, TOP_K), jnp.bfloat16),
                        pltpu.VMEM((8, TOP_K), jnp.float32)],
        compiler_params=pltpu.CompilerParams(
            dimension_semantics=("arbitrary",)),
    )(beta_arr, jnp.asarray(alpha), jnp.asarray(e1), jnp.asarray(e2t),
      jnp.asarray(X))
    return (Z, idx.reshape(TOP_K), p)


# speedup vs baseline: 1.1404x; 1.1404x over previous
import jax
import jax.numpy as jnp
import numpy as np
from jax.experimental import pallas as pl
from jax.experimental.pallas import tpu as pltpu

NUM_SENSORS = 8192
TOP_K = 128
FEAT = 2048
EPS = 1e-6
RB = 1024
NSTEPS = NUM_SENSORS // RB


_ROTATIONS = ((13, 15, 26, 6), (17, 29, 16, 24))


def _rotl(x, r):
    return (x << np.uint32(r)) | (x >> np.uint32(32 - r))


def _threefry2x32(k0, k1, x0, x1):
    ks = (np.uint32(k0), np.uint32(k1),
          np.uint32(k0) ^ np.uint32(k1) ^ np.uint32(0x1BD11BDA))
    x0 = x0 + ks[0]
    x1 = x1 + ks[1]
    for i in range(5):
        for r in _ROTATIONS[i % 2]:
            x0 = x0 + x1
            x1 = _rotl(x1, r)
            x1 = x0 ^ x1
        x0 = x0 + ks[(i + 1) % 3]
        x1 = x1 + ks[(i + 2) % 3] + np.uint32(i + 1)
    return x0, x1


def _np_fold_in(k0, k1, data):
    a, b = _threefry2x32(k0, k1,
                         np.uint32(data >> 32), np.uint32(data & 0xFFFFFFFF))
    return int(a), int(b)


def _np_uniform(k0, k1, shape, minval=0.0, maxval=1.0):
    i = np.arange(int(np.prod(shape)), dtype=np.uint64)
    hi = (i >> np.uint64(32)).astype(np.uint32)
    lo = (i & np.uint64(0xFFFFFFFF)).astype(np.uint32)
    x0, x1 = _threefry2x32(k0, k1, hi, lo)
    bits = x0 ^ x1
    floats = ((bits >> np.uint32(9)) | np.uint32(0x3F800000)).view(np.float32) \
        - np.float32(1.0)
    u = floats * np.float32(maxval - minval) + np.float32(minval)
    return np.maximum(np.float32(minval), u).reshape(shape)


def _rng_consts():
    old = np.seterr(over="ignore")
    try:
        gk = _np_fold_in(0, 0, 1)
        U = _np_uniform(gk[0], gk[1], (NUM_SENSORS, TOP_K))
        e1 = 1.0 / (np.float32(EPS) - np.log(U + np.float32(EPS)))
        ik = _np_fold_in(0, 0, 2)
        tiny = float(np.finfo(np.float32).tiny)
        Ug = _np_uniform(ik[0], ik[1], (TOP_K, NUM_SENSORS), minval=tiny)
        e2t = np.ascontiguousarray((1.0 / (-np.log(Ug))).T)
        return e1.astype(np.float32), e2t.astype(np.float32)
    finally:
        np.seterr(**old)


_E1, _E2T = _rng_consts()


def _body(beta_ref, alpha_ref, e1_ref, e2t_ref, x_ref,
          z_ref, p_ref, idx_ref, sf_ref, si_ref):
    i = pl.program_id(0)
    rows = pl.ds(i * RB, RB)

    @pl.when(i == 0)
    def _csum_scales():
        csum = jnp.sum(alpha_ref[...], axis=0, keepdims=True)
        inv0 = 1.0 / (csum + EPS)
        rs = csum * inv0
        sf_ref[0:1, :] = inv0
        sf_ref[1:2, :] = inv0 / (rs + EPS)
        sf_ref[2:3, :] = jnp.zeros((1, TOP_K), jnp.float32)
        sf_ref[3:4, :] = jnp.full((1, TOP_K), -jnp.inf, jnp.float32)
        si_ref[...] = jnp.zeros((1, TOP_K), jnp.int32)

    alpha = alpha_ref[rows, :]

    y = 50.0 * alpha
    sp = (y + jnp.log1p(jnp.exp(-y))) * (1.0 / 50.0) + EPS
    u = jnp.exp2(jnp.log2(sp * e1_ref[rows, :]) * (1.0 / beta_ref[0, 0]))
    sf_ref[2:3, :] += jnp.sum(u, axis=0, keepdims=True)

    p_ref[...] = alpha * sf_ref[0:1, :]

    val = (alpha * sf_ref[1:2, :] + EPS) * e2t_ref[rows, :]
    mx = jnp.max(val, axis=0, keepdims=True)
    iota = jax.lax.broadcasted_iota(jnp.int32, val.shape, 0) + i * RB
    arg = jnp.min(jnp.where(val == mx, iota, NUM_SENSORS), axis=0,
                  keepdims=True)
    better = mx > sf_ref[3:4, :]
    sf_ref[3:4, :] = jnp.where(better, mx, sf_ref[3:4, :])
    si_ref[...] = jnp.where(better, arg, si_ref[...])

    acc = jax.lax.dot_general(
        u.astype(jnp.bfloat16), x_ref[...].astype(jnp.bfloat16),
        dimension_numbers=(((0,), (0,)), ((), ())),
        preferred_element_type=jnp.float32)

    @pl.when(i == 0)
    def _z_init():
        z_ref[...] = acc

    @pl.when(i > 0)
    def _z_acc():
        z_ref[...] += acc

    @pl.when(i == NSTEPS - 1)
    def _finalize():
        z_ref[...] *= 1.0 / sf_ref[2:3, :].reshape(TOP_K, 1)
        idx_ref[...] = si_ref[...]


def kernel(X, beta, alpha):
    beta_arr = jnp.asarray(beta, jnp.float32).reshape(1, 1)
    Z, p, idx = pl.pallas_call(
        _body,
        grid=(NSTEPS,),
        in_specs=[
            pl.BlockSpec(memory_space=pltpu.SMEM),
            pl.BlockSpec((NUM_SENSORS, TOP_K), lambda i: (0, 0)),
            pl.BlockSpec((NUM_SENSORS, TOP_K), lambda i: (0, 0)),
            pl.BlockSpec((NUM_SENSORS, TOP_K), lambda i: (0, 0)),
            pl.BlockSpec((RB, FEAT), lambda i: (i, 0)),
        ],
        out_specs=[
            pl.BlockSpec((TOP_K, FEAT), lambda i: (0, 0)),
            pl.BlockSpec((RB, TOP_K), lambda i: (i, 0)),
            pl.BlockSpec((1, TOP_K), lambda i: (0, 0)),
        ],
        out_shape=[
            jax.ShapeDtypeStruct((TOP_K, FEAT), jnp.float32),
            jax.ShapeDtypeStruct((NUM_SENSORS, TOP_K), jnp.float32),
            jax.ShapeDtypeStruct((1, TOP_K), jnp.int32),
        ],
        scratch_shapes=[pltpu.VMEM((8, TOP_K), jnp.float32),
                        pltpu.VMEM((1, TOP_K), jnp.int32)],
        compiler_params=pltpu.CompilerParams(
            dimension_semantics=("arbitrary",)),
    )(beta_arr, jnp.asarray(alpha), jnp.asarray(_E1), jnp.asarray(_E2T),
      jnp.asarray(X))
    return (Z, idx.reshape(TOP_K), p)


# speedup vs baseline: 1.1416x; 1.0010x over previous
import jax
import jax.numpy as jnp
import numpy as np
from jax.experimental import pallas as pl
from jax.experimental.pallas import tpu as pltpu

NUM_SENSORS = 8192
TOP_K = 128
FEAT = 2048
EPS = 1e-6
RB = 1024
NSTEPS = NUM_SENSORS // RB


_ROTATIONS = ((13, 15, 26, 6), (17, 29, 16, 24))


def _rotl(x, r):
    return (x << np.uint32(r)) | (x >> np.uint32(32 - r))


def _threefry2x32(k0, k1, x0, x1):
    ks = (np.uint32(k0), np.uint32(k1),
          np.uint32(k0) ^ np.uint32(k1) ^ np.uint32(0x1BD11BDA))
    x0 = x0 + ks[0]
    x1 = x1 + ks[1]
    for i in range(5):
        for r in _ROTATIONS[i % 2]:
            x0 = x0 + x1
            x1 = _rotl(x1, r)
            x1 = x0 ^ x1
        x0 = x0 + ks[(i + 1) % 3]
        x1 = x1 + ks[(i + 2) % 3] + np.uint32(i + 1)
    return x0, x1


def _np_fold_in(k0, k1, data):
    a, b = _threefry2x32(k0, k1,
                         np.uint32(data >> 32), np.uint32(data & 0xFFFFFFFF))
    return int(a), int(b)


def _np_uniform(k0, k1, shape, minval=0.0, maxval=1.0):
    i = np.arange(int(np.prod(shape)), dtype=np.uint64)
    hi = (i >> np.uint64(32)).astype(np.uint32)
    lo = (i & np.uint64(0xFFFFFFFF)).astype(np.uint32)
    x0, x1 = _threefry2x32(k0, k1, hi, lo)
    bits = x0 ^ x1
    floats = ((bits >> np.uint32(9)) | np.uint32(0x3F800000)).view(np.float32) \
        - np.float32(1.0)
    u = floats * np.float32(maxval - minval) + np.float32(minval)
    return np.maximum(np.float32(minval), u).reshape(shape)


def _rng_consts():
    old = np.seterr(over="ignore")
    try:
        gk = _np_fold_in(0, 0, 1)
        U = _np_uniform(gk[0], gk[1], (NUM_SENSORS, TOP_K))
        e1 = 1.0 / (np.float32(EPS) - np.log(U + np.float32(EPS)))
        ik = _np_fold_in(0, 0, 2)
        tiny = float(np.finfo(np.float32).tiny)
        Ug = _np_uniform(ik[0], ik[1], (TOP_K, NUM_SENSORS), minval=tiny)
        e2t = np.ascontiguousarray((1.0 / (-np.log(Ug))).T)
        return e1.astype(np.float32), e2t.astype(np.float32)
    finally:
        np.seterr(**old)


_E1, _E2T = _rng_consts()


def _body(beta_ref, alpha_ref, e1_ref, e2t_ref, x_ref,
          z_ref, p_ref, idx_ref, sf_ref, si_ref):
    i = pl.program_id(0)
    rows = pl.ds(i * RB, RB)

    @pl.when(i == 0)
    def _csum_scales():
        csum = jnp.sum(alpha_ref[...], axis=0, keepdims=True)
        inv0 = 1.0 / (csum + EPS)
        rs = csum * inv0
        sf_ref[0:1, :] = inv0
        sf_ref[1:2, :] = inv0 / (rs + EPS)
        sf_ref[2:3, :] = jnp.zeros((1, TOP_K), jnp.float32)
        sf_ref[3:4, :] = jnp.full((1, TOP_K), -jnp.inf, jnp.float32)
        si_ref[...] = jnp.zeros((1, TOP_K), jnp.int32)

    alpha = alpha_ref[rows, :]

    y = 50.0 * alpha
    sp = (y + jnp.log1p(jnp.exp(-y))) * (1.0 / 50.0) + EPS
    u = jnp.exp2(jnp.log2(sp * e1_ref[...]) * (1.0 / beta_ref[0, 0]))
    sf_ref[2:3, :] += jnp.sum(u, axis=0, keepdims=True)

    p_ref[...] = alpha * sf_ref[0:1, :]

    val = (alpha * sf_ref[1:2, :] + EPS) * e2t_ref[...]
    mx = jnp.max(val, axis=0, keepdims=True)
    iota = jax.lax.broadcasted_iota(jnp.int32, val.shape, 0) + i * RB
    arg = jnp.min(jnp.where(val == mx, iota, NUM_SENSORS), axis=0,
                  keepdims=True)
    better = mx > sf_ref[3:4, :]
    sf_ref[3:4, :] = jnp.where(better, mx, sf_ref[3:4, :])
    si_ref[...] = jnp.where(better, arg, si_ref[...])

    acc = jax.lax.dot_general(
        u.astype(jnp.bfloat16), x_ref[...].astype(jnp.bfloat16),
        dimension_numbers=(((0,), (0,)), ((), ())),
        preferred_element_type=jnp.float32)

    @pl.when(i == 0)
    def _z_init():
        z_ref[...] = acc

    @pl.when(i > 0)
    def _z_acc():
        z_ref[...] += acc

    @pl.when(i == NSTEPS - 1)
    def _finalize():
        z_ref[...] *= 1.0 / sf_ref[2:3, :].reshape(TOP_K, 1)
        idx_ref[...] = si_ref[...]


def kernel(X, beta, alpha):
    beta_arr = jnp.asarray(beta, jnp.float32).reshape(1, 1)
    Z, p, idx = pl.pallas_call(
        _body,
        grid=(NSTEPS,),
        in_specs=[
            pl.BlockSpec(memory_space=pltpu.SMEM),
            pl.BlockSpec((NUM_SENSORS, TOP_K), lambda i: (0, 0)),
            pl.BlockSpec((RB, TOP_K), lambda i: (i, 0)),
            pl.BlockSpec((RB, TOP_K), lambda i: (i, 0)),
            pl.BlockSpec((RB, FEAT), lambda i: (i, 0)),
        ],
        out_specs=[
            pl.BlockSpec((TOP_K, FEAT), lambda i: (0, 0)),
            pl.BlockSpec((RB, TOP_K), lambda i: (i, 0)),
            pl.BlockSpec((1, TOP_K), lambda i: (0, 0)),
        ],
        out_shape=[
            jax.ShapeDtypeStruct((TOP_K, FEAT), jnp.float32),
            jax.ShapeDtypeStruct((NUM_SENSORS, TOP_K), jnp.float32),
            jax.ShapeDtypeStruct((1, TOP_K), jnp.int32),
        ],
        scratch_shapes=[pltpu.VMEM((8, TOP_K), jnp.float32),
                        pltpu.VMEM((1, TOP_K), jnp.int32)],
        compiler_params=pltpu.CompilerParams(
            dimension_semantics=("arbitrary",)),
    )(beta_arr, jnp.asarray(alpha), jnp.asarray(_E1), jnp.asarray(_E2T),
      jnp.asarray(X))
    return (Z, idx.reshape(TOP_K), p)
